# baseline (speedup 1.0000x reference)
import functools
import math

import jax
import jax.numpy as jnp
from jax import lax
from jax.experimental import pallas as pl
from jax.experimental.pallas import tpu as pltpu
from jax.experimental.pallas import tpu_sc as plsc

_N_NODES = 50_000
_N_EDGES = 3_200_000
_NW = 32
_NCHUNK = 1
_ECH = _N_EDGES // _NCHUNK
_EPW = _ECH // _NW
_BLK = 640
_NFULL = _EPW // _BLK
_TAIL = _EPW - _NFULL * _BLK
_CHUNK = 128

_S3 = math.sqrt(3.0)
_S5 = math.sqrt(5.0)


def _rsqrt(n):
    i = plsc.bitcast(n, jnp.int32)
    i = jnp.int32(0x5F3759DF) - (i >> 1)
    y = plsc.bitcast(i, jnp.float32)
    for _ in range(2):
        y = y * (jnp.float32(1.5) - jnp.float32(0.5) * n * y * y)
    return y


def _compute_block(xtab, ytab, idx_s, idx_d, z_s, z_d, hbx, hby, hbz,
                   bvx, bvy, bvz, bl, bsh, nb):
    def grp(g, carry):
        o = pl.ds(g * 16, 16)
        ns = idx_s[o]
        nd = idx_d[o]
        sx = plsc.load_gather(xtab, [ns])
        sy = plsc.load_gather(ytab, [ns])
        sz = z_s[o]
        dxr = plsc.load_gather(xtab, [nd])
        dyr = plsc.load_gather(ytab, [nd])
        dzr = z_d[o]
        vx = dxr - sx + hbx[o]
        vy = dyr - sy + hby[o]
        vz = dzr - sz + hbz[o]
        n = vx * vx + vy * vy + vz * vz
        r = _rsqrt(n)
        r = jnp.where(n > 0.0, r, jnp.float32(0.0))
        ux = vx * r
        uy = vy * r
        uz = vz * r
        bvx[o] = vx
        bvy[o] = vy
        bvz[o] = vz
        bl[o] = n * r
        s3 = jnp.float32(_S3)
        s5 = jnp.float32(_S5)
        bsh[0][o] = s3 * ux
        bsh[1][o] = s3 * uy
        bsh[2][o] = s3 * uz
        bsh[3][o] = s5 * s3 * ux * uz
        bsh[4][o] = s5 * s3 * ux * uy
        bsh[5][o] = s5 * (uy * uy - jnp.float32(0.5) * (ux * ux + uz * uz))
        bsh[6][o] = s5 * s3 * uy * uz
        bsh[7][o] = s5 * jnp.float32(0.5 * _S3) * (uz * uz - ux * ux)
        return carry

    lax.fori_loop(0, nb // 16, grp, 0)


def _do_block(ins, outs, z_sp, xtab, ytab, inbufs, obufs, sem, osem,
              base, obase, nb, drain_nb):
    srcs, dsts, shx, shy, shz = ins
    idx_s, idx_d, z_s, z_d, hbx, hby, hbz = inbufs
    bvx, bvy, bvz, bl, *bsh = obufs
    pltpu.sync_copy(srcs.at[pl.ds(base, nb)], idx_s.at[pl.ds(0, nb)])
    pltpu.sync_copy(dsts.at[pl.ds(base, nb)], idx_d.at[pl.ds(0, nb)])
    cps = []
    nfull, rem = divmod(nb, _CHUNK)
    for j in range(nfull + (1 if rem else 0)):
        o = j * _CHUNK
        c = rem if (rem and j == nfull) else _CHUNK
        cps.append(pltpu.async_copy(
            z_sp.at[idx_s.at[pl.ds(o, c)]], z_s.at[pl.ds(o, c)], sem))
        cps.append(pltpu.async_copy(
            z_sp.at[idx_d.at[pl.ds(o, c)]], z_d.at[pl.ds(o, c)], sem))
    for hbm_ref, loc_ref in zip((shx, shy, shz), (hbx, hby, hbz)):
        pltpu.sync_copy(hbm_ref.at[pl.ds(base, nb)], loc_ref.at[pl.ds(0, nb)])
    for c in cps:
        c.wait()
    _compute_block(xtab, ytab, idx_s, idx_d, z_s, z_d, hbx, hby, hbz,
                   bvx, bvy, bvz, bl, bsh, nb)
    ocps = [
        pltpu.async_copy(
            loc_ref.at[pl.ds(0, nb)], hbm_ref.at[pl.ds(obase, nb)], osem)
        for hbm_ref, loc_ref in zip(outs, obufs)
    ]
    for c in ocps:
        c.wait()


def _sc_body(chunk, xs, ys, zs, srcs, dsts, shx, shy, shz, *out_and_scratch):
    outs = out_and_scratch[:12]
    rest = out_and_scratch[12:]
    xtab, ytab, z_sp = rest[:3]
    inbufs = rest[3:10]
    obufs0 = rest[10:22]
    sem, osem = rest[22:]
    sid = lax.axis_index("s")
    cid = lax.axis_index("c")
    wid = sid * 2 + cid

    @pl.when(sid == 0)
    def _():
        pltpu.sync_copy(zs, z_sp)

    pltpu.sync_copy(xs, xtab)
    pltpu.sync_copy(ys, ytab)
    plsc.subcore_barrier()

    start = chunk * _ECH + wid * _EPW
    ostart = wid * _EPW
    ins = (srcs, dsts, shx, shy, shz)

    def blk(b, carry):
        _do_block(ins, outs, z_sp, xtab, ytab, inbufs, obufs0, sem, osem,
                  start + b * _BLK, ostart + b * _BLK, _BLK, 0)
        return carry

    lax.fori_loop(0, _NFULL, blk, 0)
    _do_block(ins, outs, z_sp, xtab, ytab, inbufs, obufs0, sem, osem,
              start + _NFULL * _BLK, ostart + _NFULL * _BLK, _TAIL, 0)


@jax.jit
def _run(xs, ys, zs, srcs, dsts, shx, shy, shz):
    mesh = plsc.VectorSubcoreMesh(core_axis_name="c", subcore_axis_name="s")
    plane = jax.ShapeDtypeStruct((_ECH,), jnp.float32)
    res = []
    for chunk in range(_NCHUNK):
        f = pl.kernel(
            functools.partial(_sc_body, chunk),
            out_type=[plane] * 12,
            mesh=mesh,
            compiler_params=pltpu.CompilerParams(needs_layout_passes=False),
            scratch_types=[
                pltpu.VMEM((_N_NODES,), jnp.float32),
                pltpu.VMEM((_N_NODES,), jnp.float32),
                pltpu.VMEM_SHARED((_N_NODES,), jnp.float32),
                pltpu.VMEM((_BLK,), jnp.int32),
                pltpu.VMEM((_BLK,), jnp.int32),
            ] + [pltpu.VMEM((_BLK,), jnp.float32)] * 17 + [
                pltpu.SemaphoreType.DMA,
                pltpu.SemaphoreType.DMA,
            ],
        )
        res.append(f(xs, ys, zs, srcs, dsts, shx, shy, shz))
    return res


def kernel(pos, edge_index, shift):
    xs = pos[:, 0]
    ys = pos[:, 1]
    zs = pos[:, 2]
    srcs = edge_index[0].astype(jnp.int32)
    dsts = edge_index[1].astype(jnp.int32)
    shx = shift[:, 0]
    shy = shift[:, 1]
    shz = shift[:, 2]
    res = _run(xs, ys, zs, srcs, dsts, shx, shy, shz)
    ev = jnp.stack(
        [jnp.concatenate([r[i] for r in res]) for i in range(3)], axis=1)
    el = jnp.concatenate([r[3] for r in res])
    esh = jnp.stack(
        [jnp.ones_like(el)]
        + [jnp.concatenate([r[i] for r in res]) for i in range(4, 12)],
        axis=1)
    return (ev, el, esh)


# speedup vs baseline: 1.2758x; 1.2758x over previous
import functools
import math

import jax
import jax.numpy as jnp
from jax import lax
from jax.experimental import pallas as pl
from jax.experimental.pallas import tpu as pltpu
from jax.experimental.pallas import tpu_sc as plsc

_N_NODES = 50_000
_N_EDGES = 3_200_000
_NW = 32
_NCHUNK = 1
_ECH = _N_EDGES // _NCHUNK
_EPW = _ECH // _NW
_BLK = 1024
_NFULL = _EPW // _BLK
_TAIL = _EPW - _NFULL * _BLK
_CHUNK = 128

_S3 = math.sqrt(3.0)
_S5 = math.sqrt(5.0)


def _rsqrt(n):
    i = plsc.bitcast(n, jnp.int32)
    i = jnp.int32(0x5F3759DF) - (i >> 1)
    y = plsc.bitcast(i, jnp.float32)
    for _ in range(2):
        y = y * (jnp.float32(1.5) - jnp.float32(0.5) * n * y * y)
    return y


def _compute_block(xtab, ytab, idx_s, idx_d, z_s, z_d, hbx, hby, hbz,
                   bvx, bvy, bvz, bl, bsh, nb):
    def grp(g, carry):
        o = pl.ds(g * 16, 16)
        ns = idx_s[o]
        nd = idx_d[o]
        sx = plsc.load_gather(xtab, [ns])
        sy = plsc.load_gather(ytab, [ns])
        sz = z_s[o]
        dxr = plsc.load_gather(xtab, [nd])
        dyr = plsc.load_gather(ytab, [nd])
        dzr = z_d[o]
        vx = dxr - sx + hbx[o]
        vy = dyr - sy + hby[o]
        vz = dzr - sz + hbz[o]
        n = vx * vx + vy * vy + vz * vz
        r = _rsqrt(n)
        r = jnp.where(n > 0.0, r, jnp.float32(0.0))
        ux = vx * r
        uy = vy * r
        uz = vz * r
        bvx[o] = vx
        bvy[o] = vy
        bvz[o] = vz
        bl[o] = n * r
        s3 = jnp.float32(_S3)
        s5 = jnp.float32(_S5)
        bsh[0][o] = s3 * ux
        bsh[1][o] = s3 * uy
        bsh[2][o] = s3 * uz
        bsh[3][o] = s5 * s3 * ux * uz
        bsh[4][o] = s5 * s3 * ux * uy
        bsh[5][o] = s5 * (uy * uy - jnp.float32(0.5) * (ux * ux + uz * uz))
        bsh[6][o] = s5 * s3 * uy * uz
        bsh[7][o] = s5 * jnp.float32(0.5 * _S3) * (uz * uz - ux * ux)
        return carry

    lax.fori_loop(0, nb // 16, grp, 0)


def _do_block(ins, outs, z_sp, xtab, ytab, inbufs, obufs, sems,
              base, obase, nb, drain_nb):
    srcs, dsts, shx, shy, shz = ins
    idx_s, idx_d, z_s, z_d, hbx, hby, hbz = inbufs
    bvx, bvy, bvz, bl, *bsh = obufs
    sem, isem, lsem, osem = sems
    icps = [
        pltpu.async_copy(
            srcs.at[pl.ds(base, nb)], idx_s.at[pl.ds(0, nb)], isem),
        pltpu.async_copy(
            dsts.at[pl.ds(base, nb)], idx_d.at[pl.ds(0, nb)], isem),
    ]
    lcps = []
    for hbm_ref, loc_ref in zip((shx, shy, shz), (hbx, hby, hbz)):
        lcps.append(pltpu.async_copy(
            hbm_ref.at[pl.ds(base, nb)], loc_ref.at[pl.ds(0, nb)], lsem))
    for c in icps:
        c.wait()
    cps = []
    nfull, rem = divmod(nb, _CHUNK)
    for j in range(nfull + (1 if rem else 0)):
        o = j * _CHUNK
        c = rem if (rem and j == nfull) else _CHUNK
        cps.append(pltpu.async_copy(
            z_sp.at[idx_s.at[pl.ds(o, c)]], z_s.at[pl.ds(o, c)], sem))
        cps.append(pltpu.async_copy(
            z_sp.at[idx_d.at[pl.ds(o, c)]], z_d.at[pl.ds(o, c)], sem))
    for c in lcps:
        c.wait()
    for c in cps:
        c.wait()
    _compute_block(xtab, ytab, idx_s, idx_d, z_s, z_d, hbx, hby, hbz,
                   bvx, bvy, bvz, bl, bsh, nb)
    ocps = [
        pltpu.async_copy(
            loc_ref.at[pl.ds(0, nb)], hbm_ref.at[pl.ds(obase, nb)], osem)
        for hbm_ref, loc_ref in zip(outs, obufs)
    ]
    for c in ocps:
        c.wait()


def _sc_body(chunk, xs, ys, zs, srcs, dsts, shx, shy, shz, *out_and_scratch):
    outs = out_and_scratch[:12]
    rest = out_and_scratch[12:]
    xtab, ytab, z_sp = rest[:3]
    inbufs = rest[3:10]
    obufs0 = rest[10:22]
    sems = rest[22:]
    sid = lax.axis_index("s")
    cid = lax.axis_index("c")
    wid = sid * 2 + cid

    @pl.when(sid == 0)
    def _():
        pltpu.sync_copy(zs, z_sp)

    pltpu.sync_copy(xs, xtab)
    pltpu.sync_copy(ys, ytab)
    plsc.subcore_barrier()

    start = chunk * _ECH + wid * _EPW
    ostart = wid * _EPW
    ins = (srcs, dsts, shx, shy, shz)

    def blk(b, carry):
        _do_block(ins, outs, z_sp, xtab, ytab, inbufs, obufs0, sems,
                  start + b * _BLK, ostart + b * _BLK, _BLK, 0)
        return carry

    lax.fori_loop(0, _NFULL, blk, 0)
    _do_block(ins, outs, z_sp, xtab, ytab, inbufs, obufs0, sems,
              start + _NFULL * _BLK, ostart + _NFULL * _BLK, _TAIL, 0)


@jax.jit
def _run(xs, ys, zs, srcs, dsts, shx, shy, shz):
    mesh = plsc.VectorSubcoreMesh(core_axis_name="c", subcore_axis_name="s")
    plane = jax.ShapeDtypeStruct((_ECH,), jnp.float32)
    res = []
    for chunk in range(_NCHUNK):
        f = pl.kernel(
            functools.partial(_sc_body, chunk),
            out_type=[plane] * 12,
            mesh=mesh,
            compiler_params=pltpu.CompilerParams(needs_layout_passes=False),
            scratch_types=[
                pltpu.VMEM((_N_NODES,), jnp.float32),
                pltpu.VMEM((_N_NODES,), jnp.float32),
                pltpu.VMEM_SHARED((_N_NODES,), jnp.float32),
                pltpu.VMEM((_BLK,), jnp.int32),
                pltpu.VMEM((_BLK,), jnp.int32),
            ] + [pltpu.VMEM((_BLK,), jnp.float32)] * 17 + [
                pltpu.SemaphoreType.DMA,
                pltpu.SemaphoreType.DMA,
                pltpu.SemaphoreType.DMA,
                pltpu.SemaphoreType.DMA,
            ],
        )
        res.append(f(xs, ys, zs, srcs, dsts, shx, shy, shz))
    return res


def kernel(pos, edge_index, shift):
    xs = pos[:, 0]
    ys = pos[:, 1]
    zs = pos[:, 2]
    srcs = edge_index[0].astype(jnp.int32)
    dsts = edge_index[1].astype(jnp.int32)
    shx = shift[:, 0]
    shy = shift[:, 1]
    shz = shift[:, 2]
    res = _run(xs, ys, zs, srcs, dsts, shx, shy, shz)
    ev = jnp.stack(
        [jnp.concatenate([r[i] for r in res]) for i in range(3)], axis=1)
    el = jnp.concatenate([r[3] for r in res])
    esh = jnp.stack(
        [jnp.ones_like(el)]
        + [jnp.concatenate([r[i] for r in res]) for i in range(4, 12)],
        axis=1)
    return (ev, el, esh)
